# baseline (speedup 1.0000x reference)
import functools

import jax
import jax.numpy as jnp
from jax import lax
from jax.experimental import pallas as pl
from jax.experimental.pallas import tpu as pltpu
from jax.experimental.pallas import tpu_sc as plsc

VOCAB = 49408
D = 512
MAX_POS = 77
N_CTX = 16
BATCH = 1024
SEQ = 61

NC, NS, L = 2, 16, 16
NW = NC * NS
B_PER_W = BATCH // NW
SEQ_PAD = 64
VPR = D // L


def _add_rows(dst, src, lo, hi):
    def body(r, carry):
        for c in range(VPR):
            sl = pl.ds(c * L, L)
            dst[r, sl] = dst[r, sl] + src[r, sl]
        return carry
    lax.fori_loop(lo, hi, body, 0)


def _sc_body(ids_hbm, table_hbm, pos_hbm, ctx_hbm, out_hbm,
             idx_v, pos_v, buf0, buf1, sem_g0, sem_g1, sem_s0, sem_s1):
    wid = lax.axis_index("s") * NC + lax.axis_index("c")
    base = wid * B_PER_W
    bufs = (buf0, buf1)
    gsems = (sem_g0, sem_g1)
    ssems = (sem_s0, sem_s1)

    pltpu.sync_copy(ids_hbm.at[pl.ds(base, B_PER_W)], idx_v)
    pltpu.sync_copy(pos_hbm, pos_v)

    for buf in bufs:
        pltpu.sync_copy(ctx_hbm, buf.at[pl.ds(0, N_CTX)])
        _add_rows(buf, pos_v, 0, N_CTX)

    def start_gather(j, p):
        pltpu.make_async_copy(
            table_hbm.at[idx_v.at[j]],
            bufs[p].at[pl.ds(N_CTX, SEQ_PAD)],
            gsems[p],
        ).start()

    def wait_gather(p):
        pltpu.make_async_copy(
            table_hbm.at[idx_v.at[0]],
            bufs[p].at[pl.ds(N_CTX, SEQ_PAD)],
            gsems[p],
        ).wait()

    def start_scatter(j, p):
        pltpu.make_async_copy(
            bufs[p].at[pl.ds(0, MAX_POS)],
            out_hbm.at[base + j],
            ssems[p],
        ).start()

    def wait_scatter(p):
        pltpu.make_async_copy(
            bufs[p].at[pl.ds(0, MAX_POS)],
            out_hbm.at[base],
            ssems[p],
        ).wait()

    start_gather(0, 0)

    def outer(g, carry):
        for b in range(2):
            j = 2 * g + b

            @pl.when(j >= 1)
            def _():
                wait_scatter(1 - b)

            @pl.when(j + 1 <= B_PER_W - 1)
            def _():
                start_gather(j + 1, 1 - b)

            wait_gather(b)
            _add_rows(bufs[b], pos_v, N_CTX, MAX_POS)
            start_scatter(j, b)
        return carry

    lax.fori_loop(0, B_PER_W // 2, outer, 0)
    wait_scatter((B_PER_W - 1) % 2)


@jax.jit
def _run(ids_pad, table, pos, ctx):
    mesh = plsc.VectorSubcoreMesh(core_axis_name="c", subcore_axis_name="s")
    f = functools.partial(
        pl.kernel,
        out_type=jax.ShapeDtypeStruct((BATCH, MAX_POS, D), jnp.float32),
        mesh=mesh,
        compiler_params=pltpu.CompilerParams(use_tc_tiling_on_sc=False),
        scratch_types=[
            pltpu.VMEM((B_PER_W, SEQ_PAD), jnp.int32),
            pltpu.VMEM((MAX_POS, D), jnp.float32),
            pltpu.VMEM((N_CTX + SEQ_PAD, D), jnp.float32),
            pltpu.VMEM((N_CTX + SEQ_PAD, D), jnp.float32),
            pltpu.SemaphoreType.DMA,
            pltpu.SemaphoreType.DMA,
            pltpu.SemaphoreType.DMA,
            pltpu.SemaphoreType.DMA,
        ],
    )(_sc_body)
    return f(ids_pad, table, pos, ctx)


def kernel(input_ids, token_embedding, position_embedding, ctx):
    ids = input_ids.astype(jnp.int32)
    ids_pad = jnp.pad(ids, ((0, 0), (0, SEQ_PAD - SEQ)))
    return _run(ids_pad, token_embedding, position_embedding, ctx)


# speedup vs baseline: 1.3174x; 1.3174x over previous
import functools

import jax
import jax.numpy as jnp
from jax import lax
from jax.experimental import pallas as pl
from jax.experimental.pallas import tpu as pltpu
from jax.experimental.pallas import tpu_sc as plsc

VOCAB = 49408
D = 512
MAX_POS = 77
N_CTX = 16
BATCH = 1024
SEQ = 61

NC, NS, L = 2, 16, 16
NW = NC * NS
B_PER_W = BATCH // NW
SEQ_PAD = 64
SLOT = N_CTX + SEQ_PAD
VPR = D // L


def _sc_body(ids_hbm, idxout_hbm, table_hbm, posx_hbm, ctx_hbm, out_hbm,
             ids_v, idxo_v, pos_v, ring, sem_g0, sem_g1, sem_s0, sem_s1):
    wid = lax.axis_index("s") * NC + lax.axis_index("c")
    base = wid * B_PER_W
    gsems = (sem_g0, sem_g1)
    ssems = (sem_s0, sem_s1)

    pltpu.sync_copy(ids_hbm.at[pl.ds(base, B_PER_W)], ids_v)
    pltpu.sync_copy(idxout_hbm.at[pl.ds(base, B_PER_W)], idxo_v)
    pltpu.sync_copy(posx_hbm.at[pl.ds(N_CTX, SEQ_PAD)], pos_v)

    pltpu.sync_copy(ctx_hbm, ring.at[pl.ds(0, N_CTX)])
    pltpu.sync_copy(posx_hbm.at[pl.ds(0, N_CTX)], ring.at[pl.ds(SLOT, N_CTX)])
    for r in range(N_CTX):
        for c in range(VPR):
            sl = pl.ds(c * L, L)
            v = ring[r, sl] + ring[SLOT + r, sl]
            ring[r, sl] = v
            ring[SLOT + r, sl] = v

    def start_gather(j, p):
        pltpu.make_async_copy(
            table_hbm.at[ids_v.at[j]],
            ring.at[pl.ds(SLOT * p + N_CTX, SEQ_PAD)],
            gsems[p],
        ).start()

    def wait_gather(p):
        pltpu.make_async_copy(
            table_hbm.at[ids_v.at[0]],
            ring.at[pl.ds(SLOT * p + N_CTX, SEQ_PAD)],
            gsems[p],
        ).wait()

    def start_scatter(j, p):
        pltpu.make_async_copy(
            ring.at[pl.ds(SLOT * p, SLOT)],
            out_hbm.at[idxo_v.at[j]],
            ssems[p],
        ).start()

    def wait_scatter(p):
        pltpu.make_async_copy(
            ring.at[pl.ds(SLOT * p, SLOT)],
            out_hbm.at[idxo_v.at[0]],
            ssems[p],
        ).wait()

    def add_pos(p):
        h = SLOT * p + N_CTX
        def body(r, carry):
            for c in range(VPR):
                sl = pl.ds(c * L, L)
                ring[h + r, sl] = ring[h + r, sl] + pos_v[r, sl]
            return carry
        lax.fori_loop(0, SEQ_PAD, body, 0)

    start_gather(0, 0)

    def outer(g, carry):
        for b in range(2):
            j = 2 * g + b

            @pl.when(j >= 1)
            def _():
                wait_scatter(1 - b)

            @pl.when(j + 1 <= B_PER_W - 1)
            def _():
                start_gather(j + 1, 1 - b)

            wait_gather(b)
            add_pos(b)
            start_scatter(j, b)
        return carry

    lax.fori_loop(0, B_PER_W // 2, outer, 0)
    wait_scatter((B_PER_W - 1) % 2)


@jax.jit
def _run(ids_ext, idx_out, table, pos_ext, ctx):
    mesh = plsc.VectorSubcoreMesh(core_axis_name="c", subcore_axis_name="s")
    f = functools.partial(
        pl.kernel,
        out_type=jax.ShapeDtypeStruct((BATCH * MAX_POS, D), jnp.float32),
        mesh=mesh,
        scratch_types=[
            pltpu.VMEM((B_PER_W, SEQ_PAD), jnp.int32),
            pltpu.VMEM((B_PER_W, SLOT), jnp.int32),
            pltpu.VMEM((SEQ_PAD, D), jnp.float32),
            pltpu.VMEM((2 * SLOT, D), jnp.float32),
            pltpu.SemaphoreType.DMA,
            pltpu.SemaphoreType.DMA,
            pltpu.SemaphoreType.DMA,
            pltpu.SemaphoreType.DMA,
        ],
    )(_sc_body)
    out = f(ids_ext, idx_out, table, pos_ext, ctx)
    return out.reshape(BATCH, MAX_POS, D)


def kernel(input_ids, token_embedding, position_embedding, ctx):
    ids = input_ids.astype(jnp.int32)
    ids_ext = jnp.concatenate(
        [ids, jnp.broadcast_to(ids[:, SEQ - 1:], (BATCH, SEQ_PAD - SEQ))],
        axis=1)
    rowbase = jnp.arange(BATCH, dtype=jnp.int32)[:, None] * MAX_POS
    ctx_cols = jnp.arange(N_CTX, dtype=jnp.int32)
    tok_cols = N_CTX + jnp.minimum(
        jnp.arange(SEQ_PAD, dtype=jnp.int32), SEQ - 1)
    idx_out = rowbase + jnp.concatenate([ctx_cols, tok_cols])[None, :]
    pos_ext = position_embedding[jnp.concatenate([ctx_cols, tok_cols])]
    return _run(ids_ext, idx_out, token_embedding, pos_ext, ctx)


# speedup vs baseline: 1.9321x; 1.4666x over previous
import functools

import jax
import jax.numpy as jnp
from jax import lax
from jax.experimental import pallas as pl
from jax.experimental.pallas import tpu as pltpu
from jax.experimental.pallas import tpu_sc as plsc

VOCAB = 49408
D = 512
MAX_POS = 77
N_CTX = 16
BATCH = 1024
SEQ = 61

NC, NS, L = 2, 16, 16
NW = NC * NS
VPR = D // L

TOK0 = N_CTX * BATCH
TPW = SEQ * BATCH // NW
CH = 64
NCH = TPW // CH
TAIL = TPW - NCH * CH
CPW = N_CTX * BATCH // NW
CREP = 32


def _sc_body(idsf_hbm, table_hbm, pos_hbm, ctx_hbm, out_hbm,
             ids_v, pos_v, ctx8, ctxrep, s0, s1,
             sem_g0, sem_g1, sem_s0, sem_s1, sem_c):
    w = lax.axis_index("s") * NC + lax.axis_index("c")
    rbase = TPW * w

    pltpu.sync_copy(idsf_hbm.at[pl.ds(rbase, TPW)], ids_v)
    pltpu.sync_copy(pos_hbm, pos_v)

    crow = w // 2
    cwin = (crow // 8) * 8
    pltpu.sync_copy(ctx_hbm.at[pl.ds(cwin, 8)], ctx8)
    crem = crow - cwin
    for c in range(VPR):
        sl = pl.ds(c * L, L)
        v = ctx8[crem, sl] + pos_v[crow, sl]
        def fill(r, carry):
            ctxrep[r, sl] = v
            return carry
        lax.fori_loop(0, CREP, fill, 0)
    for k in range(CPW // CREP):
        pltpu.make_async_copy(
            ctxrep, out_hbm.at[pl.ds(CPW * w + CREP * k, CREP)], sem_c
        ).start()

    slots = (s0, s1)
    gsems = (sem_g0, sem_g1)
    ssems = (sem_s0, sem_s1)

    def start_gather(c, p, n=CH):
        pltpu.make_async_copy(
            table_hbm.at[ids_v.at[pl.ds(CH * c, n)]],
            slots[p].at[pl.ds(0, n)],
            gsems[p],
        ).start()

    def wait_gather(p, n=CH):
        pltpu.make_async_copy(
            table_hbm.at[ids_v.at[pl.ds(0, n)]],
            slots[p].at[pl.ds(0, n)],
            gsems[p],
        ).wait()

    def start_scatter(c, p, n=CH):
        pltpu.make_async_copy(
            slots[p].at[pl.ds(0, n)],
            out_hbm.at[pl.ds(TOK0 + rbase + CH * c, n)],
            ssems[p],
        ).start()

    def wait_scatter(p, n=CH):
        pltpu.make_async_copy(
            slots[p].at[pl.ds(0, n)],
            out_hbm.at[pl.ds(TOK0, n)],
            ssems[p],
        ).wait()

    def add_pos(c, p, n=CH):
        gbase = rbase + CH * c
        def row(r, carry):
            prow = N_CTX + (gbase + r) // BATCH
            for cc in range(VPR):
                sl = pl.ds(cc * L, L)
                slots[p][r, sl] = slots[p][r, sl] + pos_v[prow, sl]
            return carry
        lax.fori_loop(0, n, row, 0)

    start_gather(0, 0)

    def outer(m, carry):
        for b in range(2):
            c = 2 * m + b

            @pl.when(c >= 1)
            def _():
                wait_scatter(1 - b)

            @pl.when(c + 1 <= NCH - 1)
            def _():
                start_gather(c + 1, 1 - b)

            wait_gather(b)
            add_pos(c, b)
            start_scatter(c, b)
        return carry

    lax.fori_loop(0, NCH // 2, outer, 0)

    start_gather(NCH, 0, TAIL)
    wait_scatter(1)
    wait_gather(0, TAIL)
    add_pos(NCH, 0, TAIL)
    start_scatter(NCH, 0, TAIL)
    wait_scatter(0, TAIL)
    for _ in range(CPW // CREP):
        pltpu.make_async_copy(
            ctxrep, out_hbm.at[pl.ds(0, CREP)], sem_c
        ).wait()


@jax.jit
def _run(ids_flat, table, pos, ctx):
    mesh = plsc.VectorSubcoreMesh(core_axis_name="c", subcore_axis_name="s")
    f = functools.partial(
        pl.kernel,
        out_type=jax.ShapeDtypeStruct((MAX_POS * BATCH, D), jnp.float32),
        mesh=mesh,
        scratch_types=[
            pltpu.VMEM((TPW,), jnp.int32),
            pltpu.VMEM((MAX_POS, D), jnp.float32),
            pltpu.VMEM((8, D), jnp.float32),
            pltpu.VMEM((CREP, D), jnp.float32),
            pltpu.VMEM((CH, D), jnp.float32),
            pltpu.VMEM((CH, D), jnp.float32),
            pltpu.SemaphoreType.DMA,
            pltpu.SemaphoreType.DMA,
            pltpu.SemaphoreType.DMA,
            pltpu.SemaphoreType.DMA,
            pltpu.SemaphoreType.DMA,
        ],
    )(_sc_body)
    out = f(ids_flat, table, pos, ctx)
    return out.reshape(MAX_POS, BATCH, D).transpose(1, 0, 2)


def kernel(input_ids, token_embedding, position_embedding, ctx):
    ids_flat = input_ids.astype(jnp.int32).T.reshape(-1)
    return _run(ids_flat, token_embedding, position_embedding, ctx)


# speedup vs baseline: 4.3771x; 2.2654x over previous
import functools

import jax
import jax.numpy as jnp
from jax import lax
from jax.experimental import pallas as pl
from jax.experimental.pallas import tpu as pltpu
from jax.experimental.pallas import tpu_sc as plsc

VOCAB = 49408
D = 512
MAX_POS = 77
N_CTX = 16
BATCH = 1024
SEQ = 61

NC, NS, L = 2, 16, 16
NW = NC * NS
VPR = D // L

TOK0 = N_CTX * BATCH
TPW = SEQ * BATCH // NW
CH = 32
NCH = TPW // CH
CPW = N_CTX * BATCH // NW
CREP = 32


def _sc_body(idsf_hbm, table_hbm, pos_hbm, ctx_hbm, out_hbm,
             ids_v, pos_v, ctx8, ctxrep, s0, s1,
             sem_g0, sem_g1, sem_s0, sem_s1, sem_c):
    w = lax.axis_index("s") * NC + lax.axis_index("c")
    rbase = TPW * w

    pltpu.sync_copy(idsf_hbm.at[pl.ds(rbase, TPW)], ids_v)
    pltpu.sync_copy(pos_hbm, pos_v)

    crow = w // 2
    cwin = (crow // 8) * 8
    pltpu.sync_copy(ctx_hbm.at[pl.ds(cwin, 8)], ctx8)
    crem = crow - cwin
    for c in range(VPR):
        sl = pl.ds(c * L, L)
        v = ctx8[crem, sl] + pos_v[crow, sl]
        def fill(r, carry):
            ctxrep[r, sl] = v
            return carry
        lax.fori_loop(0, CREP, fill, 0)
    for k in range(CPW // CREP):
        pltpu.make_async_copy(
            ctxrep, out_hbm.at[pl.ds(CPW * w + CREP * k, CREP)], sem_c
        ).start()

    slots = (s0, s1)
    gsems = (sem_g0, sem_g1)
    ssems = (sem_s0, sem_s1)

    def start_gather(c, p, n=CH):
        pltpu.make_async_copy(
            table_hbm.at[ids_v.at[pl.ds(CH * c, n)]],
            slots[p].at[pl.ds(0, n)],
            gsems[p],
        ).start()

    def wait_gather(p, n=CH):
        pltpu.make_async_copy(
            table_hbm.at[ids_v.at[pl.ds(0, n)]],
            slots[p].at[pl.ds(0, n)],
            gsems[p],
        ).wait()

    def start_scatter(c, p, n=CH):
        pltpu.make_async_copy(
            slots[p].at[pl.ds(0, n)],
            out_hbm.at[pl.ds(TOK0 + rbase + CH * c, n)],
            ssems[p],
        ).start()

    def wait_scatter(p, n=CH):
        pltpu.make_async_copy(
            slots[p].at[pl.ds(0, n)],
            out_hbm.at[pl.ds(TOK0, n)],
            ssems[p],
        ).wait()

    def add_pos(c, p, n=CH):
        prow = N_CTX + (rbase + CH * c) // BATCH
        for cc in range(VPR):
            sl = pl.ds(cc * L, L)
            pv = pos_v[prow, sl]

            @plsc.parallel_loop(0, n, step=1, unroll=8)
            def _(r):
                slots[p][r, sl] = slots[p][r, sl] + pv

    start_gather(0, 0)

    def outer(m, carry):
        for b in range(2):
            c = 2 * m + b

            @pl.when(c >= 1)
            def _():
                wait_scatter(1 - b)

            @pl.when(c + 1 <= NCH - 1)
            def _():
                start_gather(c + 1, 1 - b)

            wait_gather(b)
            add_pos(c, b)
            start_scatter(c, b)
        return carry

    lax.fori_loop(0, (NCH - 1) // 2, outer, 0)

    wait_gather(0)
    add_pos(NCH - 1, 0)
    start_scatter(NCH - 1, 0)
    wait_scatter(1)
    wait_scatter(0)
    for _ in range(CPW // CREP):
        pltpu.make_async_copy(
            ctxrep, out_hbm.at[pl.ds(0, CREP)], sem_c
        ).wait()


@jax.jit
def _run(ids_flat, table, pos, ctx):
    mesh = plsc.VectorSubcoreMesh(core_axis_name="c", subcore_axis_name="s")
    f = functools.partial(
        pl.kernel,
        out_type=jax.ShapeDtypeStruct((MAX_POS * BATCH, D), jnp.float32),
        mesh=mesh,
        scratch_types=[
            pltpu.VMEM((TPW,), jnp.int32),
            pltpu.VMEM((MAX_POS, D), jnp.float32),
            pltpu.VMEM((8, D), jnp.float32),
            pltpu.VMEM((CREP, D), jnp.float32),
            pltpu.VMEM((CH, D), jnp.float32),
            pltpu.VMEM((CH, D), jnp.float32),
            pltpu.SemaphoreType.DMA,
            pltpu.SemaphoreType.DMA,
            pltpu.SemaphoreType.DMA,
            pltpu.SemaphoreType.DMA,
            pltpu.SemaphoreType.DMA,
        ],
    )(_sc_body)
    out = f(ids_flat, table, pos, ctx)
    return out.reshape(MAX_POS, BATCH, D).transpose(1, 0, 2)


def kernel(input_ids, token_embedding, position_embedding, ctx):
    ids_flat = input_ids.astype(jnp.int32).T.reshape(-1)
    return _run(ids_flat, token_embedding, position_embedding, ctx)
